# Initial kernel scaffold; baseline (speedup 1.0000x reference)
import jax
import jax.numpy as jnp
from jax.experimental import pallas as pl


def kernel(edge_attr, table):
    raise NotImplementedError("write your pallas kernel here")


# speedup vs baseline: 7.0381x; 7.0381x over previous
import functools

import jax
import jax.numpy as jnp
from jax import lax
from jax.experimental import pallas as pl
from jax.experimental.pallas import tpu as pltpu
from jax.experimental.pallas import tpu_sc as plsc

EDGE_ATTR_DIM = 512
HIDDEN_DIM = 16
N_EDGES = 3_200_000

NC = 2
NS = 16
NW = NC * NS

B_PER_W = N_EDGES // NW
CHUNK = 2000
NSTEP = B_PER_W // CHUNK


def _make_gather():
    mesh = plsc.VectorSubcoreMesh(
        core_axis_name="c", subcore_axis_name="s", num_cores=NC, num_subcores=NS
    )

    @functools.partial(
        pl.kernel,
        out_type=jax.ShapeDtypeStruct((N_EDGES, HIDDEN_DIM), jnp.float32),
        mesh=mesh,
        scratch_types=[
            pltpu.VMEM((CHUNK,), jnp.int32),
            pltpu.VMEM((CHUNK, HIDDEN_DIM), jnp.float32),
            pltpu.SemaphoreType.DMA,
        ],
        compiler_params=pltpu.CompilerParams(use_tc_tiling_on_sc=False),
    )
    def gather_kernel(table_hbm, idx_hbm, out_hbm, idx_v, rows_v, sem):
        wid = lax.axis_index("s") * NC + lax.axis_index("c")
        wbase = wid * B_PER_W

        def body(step, carry):
            base = wbase + step * CHUNK
            pltpu.sync_copy(idx_hbm.at[pl.ds(base, CHUNK)], idx_v)
            pltpu.async_copy(table_hbm.at[idx_v], rows_v, sem).wait()
            pltpu.sync_copy(rows_v, out_hbm.at[pl.ds(base, CHUNK)])
            return carry

        lax.fori_loop(0, NSTEP, body, 0)

    return gather_kernel


@functools.lru_cache(maxsize=1)
def _gather():
    return _make_gather()


def kernel(edge_attr, table):
    return _gather()(table, edge_attr)


# speedup vs baseline: 7.0619x; 1.0034x over previous
import functools

import jax
import jax.numpy as jnp
from jax import lax
from jax.experimental import pallas as pl
from jax.experimental.pallas import tpu as pltpu
from jax.experimental.pallas import tpu_sc as plsc

EDGE_ATTR_DIM = 512
HIDDEN_DIM = 16
N_EDGES = 3_200_000

NC = 2
NS = 16
NW = NC * NS

B_PER_W = N_EDGES // NW
CHUNK = 2000
NSTEP = B_PER_W // CHUNK
NBUF = 2


def _make_gather():
    mesh = plsc.VectorSubcoreMesh(
        core_axis_name="c", subcore_axis_name="s", num_cores=NC, num_subcores=NS
    )

    @functools.partial(
        pl.kernel,
        out_type=jax.ShapeDtypeStruct((N_EDGES, HIDDEN_DIM), jnp.float32),
        mesh=mesh,
        scratch_types=[
            pltpu.VMEM((NBUF, CHUNK), jnp.int32),
            pltpu.VMEM((NBUF, CHUNK, HIDDEN_DIM), jnp.float32),
            [pltpu.SemaphoreType.DMA] * NBUF,
            [pltpu.SemaphoreType.DMA] * NBUF,
            [pltpu.SemaphoreType.DMA] * NBUF,
        ],
        compiler_params=pltpu.CompilerParams(use_tc_tiling_on_sc=False),
    )
    def gather_kernel(table_hbm, idx_hbm, out_hbm, idx_v, rows_v, isems, gsems, osems):
        wid = lax.axis_index("s") * NC + lax.axis_index("c")
        wbase = wid * B_PER_W

        def start_idx(step, b):
            base = wbase + step * CHUNK
            pltpu.async_copy(idx_hbm.at[pl.ds(base, CHUNK)], idx_v.at[b], isems[b])

        def wait_idx(b):
            pltpu.make_async_copy(
                idx_hbm.at[pl.ds(0, CHUNK)], idx_v.at[b], isems[b]
            ).wait()

        def start_gather(b):
            pltpu.async_copy(table_hbm.at[idx_v.at[b]], rows_v.at[b], gsems[b])

        def wait_gather(b):
            pltpu.make_async_copy(
                table_hbm.at[idx_v.at[b]], rows_v.at[b], gsems[b]
            ).wait()

        def start_store(step, b):
            base = wbase + step * CHUNK
            pltpu.async_copy(
                rows_v.at[b], out_hbm.at[pl.ds(base, CHUNK)], osems[b]
            )

        def wait_store(b):
            pltpu.make_async_copy(
                rows_v.at[b], out_hbm.at[pl.ds(0, CHUNK)], osems[b]
            ).wait()

        for b in range(NBUF):
            start_idx(b, b)

        def body(i, carry):
            for b in range(NBUF):
                step = i * NBUF + b

                @pl.when(step >= NBUF)
                def _():
                    wait_store(b)

                wait_idx(b)
                start_gather(b)
                wait_gather(b)

                @pl.when(step + NBUF < NSTEP)
                def _():
                    start_idx(step + NBUF, b)

                start_store(step, b)
            return carry

        lax.fori_loop(0, NSTEP // NBUF, body, 0)

        for b in range(NBUF):
            wait_store(b)

    return gather_kernel


@functools.lru_cache(maxsize=1)
def _gather():
    return _make_gather()


def kernel(edge_attr, table):
    return _gather()(table, edge_attr)


# speedup vs baseline: 7.0682x; 1.0009x over previous
import functools

import jax
import jax.numpy as jnp
from jax import lax
from jax.experimental import pallas as pl
from jax.experimental.pallas import tpu as pltpu
from jax.experimental.pallas import tpu_sc as plsc

EDGE_ATTR_DIM = 512
HIDDEN_DIM = 16
N_EDGES = 3_200_000

NC = 2
NS = 16
NW = NC * NS

B_PER_W = N_EDGES // NW
CHUNK = 1000
NSTEP = B_PER_W // CHUNK
NBUF = 4
AHEAD = NBUF - 1


def _make_gather():
    mesh = plsc.VectorSubcoreMesh(
        core_axis_name="c", subcore_axis_name="s", num_cores=NC, num_subcores=NS
    )

    @functools.partial(
        pl.kernel,
        out_type=jax.ShapeDtypeStruct((N_EDGES, HIDDEN_DIM), jnp.float32),
        mesh=mesh,
        scratch_types=[
            pltpu.VMEM((NBUF, CHUNK), jnp.int32),
            pltpu.VMEM((NBUF, CHUNK, HIDDEN_DIM), jnp.float32),
            [pltpu.SemaphoreType.DMA] * NBUF,
            [pltpu.SemaphoreType.DMA] * NBUF,
            [pltpu.SemaphoreType.DMA] * NBUF,
        ],
        compiler_params=pltpu.CompilerParams(use_tc_tiling_on_sc=False),
    )
    def gather_kernel(table_hbm, idx_hbm, out_hbm, idx_v, rows_v, isems, gsems, osems):
        wid = lax.axis_index("s") * NC + lax.axis_index("c")
        wbase = wid * B_PER_W

        def start_idx(step, b):
            base = wbase + step * CHUNK
            pltpu.async_copy(idx_hbm.at[pl.ds(base, CHUNK)], idx_v.at[b], isems[b])

        def wait_idx(b):
            pltpu.make_async_copy(
                idx_hbm.at[pl.ds(0, CHUNK)], idx_v.at[b], isems[b]
            ).wait()

        def start_gather(b):
            pltpu.async_copy(table_hbm.at[idx_v.at[b]], rows_v.at[b], gsems[b])

        def wait_gather(b):
            pltpu.make_async_copy(
                table_hbm.at[idx_v.at[b]], rows_v.at[b], gsems[b]
            ).wait()

        def start_store(step, b):
            base = wbase + step * CHUNK
            pltpu.async_copy(
                rows_v.at[b], out_hbm.at[pl.ds(base, CHUNK)], osems[b]
            )

        def wait_store(b):
            pltpu.make_async_copy(
                rows_v.at[b], out_hbm.at[pl.ds(0, CHUNK)], osems[b]
            ).wait()

        for b in range(NBUF):
            start_idx(b, b)
        for j in range(AHEAD):
            wait_idx(j)
            start_gather(j)

        def body(i, carry):
            for b in range(NBUF):
                step = i * NBUF + b
                nb = (b + AHEAD) % NBUF

                @pl.when(step + AHEAD < NSTEP)
                def _():
                    @pl.when(step + AHEAD >= NBUF)
                    def _():
                        wait_store(nb)
                    wait_idx(nb)
                    start_gather(nb)

                wait_gather(b)
                start_store(step, b)

                @pl.when(step + NBUF < NSTEP)
                def _():
                    start_idx(step + NBUF, b)

            return carry

        lax.fori_loop(0, NSTEP // NBUF, body, 0)

        for b in range(NBUF):
            wait_store(b)

    return gather_kernel


@functools.lru_cache(maxsize=1)
def _gather():
    return _make_gather()


def kernel(edge_attr, table):
    return _gather()(table, edge_attr)


# speedup vs baseline: 7.3946x; 1.0462x over previous
import functools

import jax
import jax.numpy as jnp
from jax import lax
from jax.experimental import pallas as pl
from jax.experimental.pallas import tpu as pltpu
from jax.experimental.pallas import tpu_sc as plsc

EDGE_ATTR_DIM = 512
HIDDEN_DIM = 16
N_EDGES = 3_200_000

NC = 2
NS = 16
NW = NC * NS
LANES = 16

B_PER_W = N_EDGES // NW
CHUNK = 2000
NSTEP = B_PER_W // CHUNK
NBUF = 2

_SPLAT_DN = lax.GatherDimensionNumbers(
    offset_dims=(), collapsed_slice_dims=(0,), start_index_map=(0,)
)


def _lane_splat(vec, lane):
    idx = jnp.full((LANES, 1), lane, dtype=jnp.int32)
    return lax.gather(
        vec, idx, _SPLAT_DN, (1,),
        mode=lax.GatherScatterMode.PROMISE_IN_BOUNDS,
    )


def _make_gather():
    mesh = plsc.VectorSubcoreMesh(
        core_axis_name="c", subcore_axis_name="s", num_cores=NC, num_subcores=NS
    )

    @functools.partial(
        pl.kernel,
        out_type=jax.ShapeDtypeStruct((N_EDGES, HIDDEN_DIM), jnp.float32),
        mesh=mesh,
        scratch_types=[
            pltpu.VMEM((EDGE_ATTR_DIM, HIDDEN_DIM), jnp.float32),
            pltpu.VMEM((NBUF, CHUNK), jnp.int32),
            pltpu.VMEM((NBUF, CHUNK, HIDDEN_DIM), jnp.float32),
            [pltpu.SemaphoreType.DMA] * NBUF,
            [pltpu.SemaphoreType.DMA] * NBUF,
        ],
        compiler_params=pltpu.CompilerParams(
            use_tc_tiling_on_sc=False, needs_layout_passes=False
        ),
    )
    def gather_kernel(table_hbm, idx_hbm, out_hbm, table_v, idx_v, rows_v,
                      isems, osems):
        wid = lax.axis_index("s") * NC + lax.axis_index("c")
        wbase = wid * B_PER_W

        def start_idx(step, b):
            base = wbase + step * CHUNK
            pltpu.async_copy(idx_hbm.at[pl.ds(base, CHUNK)], idx_v.at[b], isems[b])

        def wait_idx(b):
            pltpu.make_async_copy(
                idx_hbm.at[pl.ds(0, CHUNK)], idx_v.at[b], isems[b]
            ).wait()

        def start_store(step, b):
            base = wbase + step * CHUNK
            pltpu.async_copy(
                rows_v.at[b], out_hbm.at[pl.ds(base, CHUNK)], osems[b]
            )

        def wait_store(b):
            pltpu.make_async_copy(
                rows_v.at[b], out_hbm.at[pl.ds(0, CHUNK)], osems[b]
            ).wait()

        pltpu.sync_copy(table_hbm, table_v)

        col_iota = lax.iota(jnp.int32, LANES)

        for b in range(NBUF):
            start_idx(b, b)

        def body(i, carry):
            for b in range(NBUF):
                step = i * NBUF + b

                @pl.when(step >= NBUF)
                def _():
                    wait_store(b)

                wait_idx(b)

                def blk(j, carry):
                    iv = idx_v[b, pl.ds(j * LANES, LANES)]
                    for l in range(LANES):
                        row = plsc.load_gather(
                            table_v, [_lane_splat(iv, l), col_iota]
                        )
                        rows_v[b, j * LANES + l, :] = row
                    return carry

                lax.fori_loop(0, CHUNK // LANES, blk, 0)

                start_store(step, b)

                @pl.when(step + NBUF < NSTEP)
                def _():
                    start_idx(step + NBUF, b)

            return carry

        lax.fori_loop(0, NSTEP // NBUF, body, 0)

        for b in range(NBUF):
            wait_store(b)

    return gather_kernel


@functools.lru_cache(maxsize=1)
def _gather():
    return _make_gather()


def kernel(edge_attr, table):
    return _gather()(table, edge_attr)


# speedup vs baseline: 7.6116x; 1.0294x over previous
import functools

import jax
import jax.numpy as jnp
from jax import lax
from jax.experimental import pallas as pl
from jax.experimental.pallas import tpu as pltpu
from jax.experimental.pallas import tpu_sc as plsc

EDGE_ATTR_DIM = 512
HIDDEN_DIM = 16
N_EDGES = 3_200_000

NC = 2
NS = 16
NW = NC * NS
LANES = 16

CHUNK = 1024
NCHUNKS = N_EDGES // CHUNK
NK_BASE = NCHUNKS // NW
NK_REM = NCHUNKS % NW
NBUF = 2

_SPLAT_DN = lax.GatherDimensionNumbers(
    offset_dims=(), collapsed_slice_dims=(0,), start_index_map=(0,)
)


def _lane_splat(vec, lane):
    idx = jnp.full((LANES, 1), lane, dtype=jnp.int32)
    return lax.gather(
        vec, idx, _SPLAT_DN, (1,),
        mode=lax.GatherScatterMode.PROMISE_IN_BOUNDS,
    )


def _make_gather():
    mesh = plsc.VectorSubcoreMesh(
        core_axis_name="c", subcore_axis_name="s", num_cores=NC, num_subcores=NS
    )

    @functools.partial(
        pl.kernel,
        out_type=jax.ShapeDtypeStruct((N_EDGES, HIDDEN_DIM), jnp.float32),
        mesh=mesh,
        scratch_types=[
            pltpu.VMEM((EDGE_ATTR_DIM, HIDDEN_DIM), jnp.float32),
            pltpu.VMEM((NBUF, CHUNK), jnp.int32),
            pltpu.VMEM((NBUF, CHUNK, HIDDEN_DIM), jnp.float32),
            [pltpu.SemaphoreType.DMA] * NBUF,
            [pltpu.SemaphoreType.DMA] * NBUF,
        ],
        compiler_params=pltpu.CompilerParams(
            use_tc_tiling_on_sc=False, needs_layout_passes=False
        ),
    )
    def gather_kernel(table_hbm, idx_hbm, out_hbm, table_v, idx_v, rows_v,
                      isems, osems):
        wid = lax.axis_index("s") * NC + lax.axis_index("c")
        nk = NK_BASE + (wid < NK_REM).astype(jnp.int32)

        def start_idx(step, b):
            base = (wid + step * NW) * CHUNK
            pltpu.async_copy(idx_hbm.at[pl.ds(base, CHUNK)], idx_v.at[b], isems[b])

        def wait_idx(b):
            pltpu.make_async_copy(
                idx_hbm.at[pl.ds(0, CHUNK)], idx_v.at[b], isems[b]
            ).wait()

        def start_store(step, b):
            base = (wid + step * NW) * CHUNK
            pltpu.async_copy(
                rows_v.at[b], out_hbm.at[pl.ds(base, CHUNK)], osems[b]
            )

        def wait_store(b):
            pltpu.make_async_copy(
                rows_v.at[b], out_hbm.at[pl.ds(0, CHUNK)], osems[b]
            ).wait()

        pltpu.sync_copy(table_hbm, table_v)

        col_iota = lax.iota(jnp.int32, LANES)
        zero16 = jnp.zeros((LANES,), jnp.int32)

        def compute(b):
            def blk(j, carry):
                for u in range(2):
                    base = (2 * j + u) * LANES
                    iv16 = idx_v[b, pl.ds(base, LANES)] << 4
                    for l in range(LANES):
                        addr = _lane_splat(iv16, l) | col_iota
                        row = plsc.load_gather(table_v, [zero16, addr])
                        rows_v[b, base + l, :] = row
                return carry

            lax.fori_loop(0, CHUNK // (2 * LANES), blk, 0)

        def do_step(step, b):
            @pl.when(step >= NBUF)
            def _():
                wait_store(b)

            wait_idx(b)
            compute(b)
            start_store(step, b)

            @pl.when(step + NBUF < nk)
            def _():
                start_idx(step + NBUF, b)

        for b in range(NBUF):
            start_idx(b, b)

        def body(i, carry):
            do_step(2 * i, 0)
            do_step(2 * i + 1, 1)
            return carry

        lax.fori_loop(0, nk // 2, body, 0)

        @pl.when(nk % 2 == 1)
        def _():
            do_step(nk - 1, 0)

        for b in range(NBUF):
            wait_store(b)

    return gather_kernel


@functools.lru_cache(maxsize=1)
def _gather():
    return _make_gather()


def kernel(edge_attr, table):
    return _gather()(table, edge_attr)


# speedup vs baseline: 29.1158x; 3.8252x over previous
import functools

import jax
import jax.numpy as jnp
from jax import lax
from jax.experimental import pallas as pl
from jax.experimental.pallas import tpu as pltpu
from jax.experimental.pallas import tpu_sc as plsc

EDGE_ATTR_DIM = 512
HIDDEN_DIM = 16
N_EDGES = 3_200_000

NC = 2
NS = 16
NW = NC * NS
LANES = 16

CHUNK = 1024
TPC = CHUNK // 128
NTILES = N_EDGES // 128
NCHUNKS = N_EDGES // CHUNK
NK_BASE = NCHUNKS // NW
NK_REM = NCHUNKS % NW
NBUF = 2


def _make_gather():
    mesh = plsc.VectorSubcoreMesh(
        core_axis_name="c", subcore_axis_name="s", num_cores=NC, num_subcores=NS
    )

    @functools.partial(
        pl.kernel,
        out_type=jax.ShapeDtypeStruct((2, NTILES, 8, 128), jnp.float32),
        mesh=mesh,
        scratch_types=[
            pltpu.VMEM((HIDDEN_DIM, EDGE_ATTR_DIM), jnp.float32),
            pltpu.VMEM((NBUF, CHUNK), jnp.int32),
            pltpu.VMEM((NBUF, TPC, 8, 128), jnp.float32),
            pltpu.VMEM((NBUF, TPC, 8, 128), jnp.float32),
            [pltpu.SemaphoreType.DMA] * NBUF,
            [pltpu.SemaphoreType.DMA] * NBUF,
            [pltpu.SemaphoreType.DMA] * NBUF,
        ],
        compiler_params=pltpu.CompilerParams(
            use_tc_tiling_on_sc=False, needs_layout_passes=False
        ),
    )
    def gather_kernel(tab_t_hbm, idx_hbm, out_hbm, tab_v, idx_v, buf0, buf1,
                      isems, osems0, osems1):
        wid = lax.axis_index("s") * NC + lax.axis_index("c")
        nk = NK_BASE + (wid < NK_REM).astype(jnp.int32)

        def start_idx(step, b):
            base = (wid + step * NW) * CHUNK
            pltpu.async_copy(idx_hbm.at[pl.ds(base, CHUNK)], idx_v.at[b], isems[b])

        def wait_idx(b):
            pltpu.make_async_copy(
                idx_hbm.at[pl.ds(0, CHUNK)], idx_v.at[b], isems[b]
            ).wait()

        def start_store(step, b):
            tb = (wid + step * NW) * TPC
            pltpu.async_copy(
                buf0.at[b], out_hbm.at[0, pl.ds(tb, TPC)], osems0[b]
            )
            pltpu.async_copy(
                buf1.at[b], out_hbm.at[1, pl.ds(tb, TPC)], osems1[b]
            )

        def wait_store(b):
            pltpu.make_async_copy(
                buf0.at[b], out_hbm.at[0, pl.ds(0, TPC)], osems0[b]
            ).wait()
            pltpu.make_async_copy(
                buf1.at[b], out_hbm.at[1, pl.ds(0, TPC)], osems1[b]
            ).wait()

        pltpu.sync_copy(tab_t_hbm, tab_v)

        def compute(b):
            def tile_blk(t, carry):
                d0 = buf0.at[b].at[t]
                d1 = buf1.at[b].at[t]
                for j in range(8):
                    iv = idx_v[b, pl.ds(t * 128 + j * LANES, LANES)]
                    for c in range(HIDDEN_DIM):
                        vec = plsc.load_gather(tab_v.at[c], [iv])
                        dst = d0 if c < 8 else d1
                        dst[c % 8, pl.ds(j * LANES, LANES)] = vec
                return carry

            lax.fori_loop(0, TPC, tile_blk, 0)

        def do_step(step, b):
            @pl.when(step >= NBUF)
            def _():
                wait_store(b)

            wait_idx(b)
            compute(b)
            start_store(step, b)

            @pl.when(step + NBUF < nk)
            def _():
                start_idx(step + NBUF, b)

        for b in range(NBUF):
            start_idx(b, b)

        def body(i, carry):
            do_step(2 * i, 0)
            do_step(2 * i + 1, 1)
            return carry

        lax.fori_loop(0, nk // 2, body, 0)

        @pl.when(nk % 2 == 1)
        def _():
            do_step(nk - 1, 0)

        for b in range(NBUF):
            wait_store(b)

    return gather_kernel


@functools.lru_cache(maxsize=1)
def _gather():
    return _make_gather()


def kernel(edge_attr, table):
    out_phys = _gather()(table.T, edge_attr)
    return out_phys.transpose(1, 3, 0, 2).reshape(N_EDGES, HIDDEN_DIM)


# speedup vs baseline: 80.8627x; 2.7773x over previous
import functools

import jax
import jax.numpy as jnp
from jax import lax
from jax.experimental import pallas as pl
from jax.experimental.pallas import tpu as pltpu
from jax.experimental.pallas import tpu_sc as plsc

EDGE_ATTR_DIM = 512
HIDDEN_DIM = 16
N_EDGES = 3_200_000

NC = 2
NS = 16
NW = NC * NS
LANES = 16

CHUNK = 1024
TPC = CHUNK // 128
NTILES = N_EDGES // 128
NCHUNKS = N_EDGES // CHUNK
NK_BASE = NCHUNKS // NW
NK_REM = NCHUNKS % NW
NBUF = 2


def _make_gather():
    mesh = plsc.VectorSubcoreMesh(
        core_axis_name="c", subcore_axis_name="s", num_cores=NC, num_subcores=NS
    )

    @functools.partial(
        pl.kernel,
        out_type=jax.ShapeDtypeStruct((2, NTILES, 8, 128), jnp.float32),
        mesh=mesh,
        scratch_types=[
            pltpu.VMEM((HIDDEN_DIM, EDGE_ATTR_DIM), jnp.float32),
            pltpu.VMEM((NBUF, CHUNK), jnp.int32),
            pltpu.VMEM((NBUF, TPC, 8, 128), jnp.float32),
            pltpu.VMEM((NBUF, TPC, 8, 128), jnp.float32),
            [pltpu.SemaphoreType.DMA] * NBUF,
            [pltpu.SemaphoreType.DMA] * NBUF,
            [pltpu.SemaphoreType.DMA] * NBUF,
        ],
        compiler_params=pltpu.CompilerParams(
            use_tc_tiling_on_sc=False, needs_layout_passes=False
        ),
    )
    def gather_kernel(tab_t_hbm, idx_hbm, out_hbm, tab_v, idx_v, buf0, buf1,
                      isems, osems0, osems1):
        wid = lax.axis_index("s") * NC + lax.axis_index("c")
        nk = NK_BASE + (wid < NK_REM).astype(jnp.int32)

        def start_idx(step, b):
            base = (wid + step * NW) * CHUNK
            pltpu.async_copy(idx_hbm.at[pl.ds(base, CHUNK)], idx_v.at[b], isems[b])

        def wait_idx(b):
            pltpu.make_async_copy(
                idx_hbm.at[pl.ds(0, CHUNK)], idx_v.at[b], isems[b]
            ).wait()

        def start_store(step, b):
            tb = (wid + step * NW) * TPC
            pltpu.async_copy(
                buf0.at[b], out_hbm.at[0, pl.ds(tb, TPC)], osems0[b]
            )
            pltpu.async_copy(
                buf1.at[b], out_hbm.at[1, pl.ds(tb, TPC)], osems1[b]
            )

        def wait_store(b):
            pltpu.make_async_copy(
                buf0.at[b], out_hbm.at[0, pl.ds(0, TPC)], osems0[b]
            ).wait()
            pltpu.make_async_copy(
                buf1.at[b], out_hbm.at[1, pl.ds(0, TPC)], osems1[b]
            ).wait()

        pltpu.sync_copy(tab_t_hbm, tab_v)

        def compute(b):
            def tile_blk(t, carry):
                d0 = buf0.at[b].at[t]
                d1 = buf1.at[b].at[t]

                ivs = [
                    idx_v[b, pl.ds(t * 128 + j * LANES, LANES)]
                    for j in range(8)
                ]

                def store(j, c, vec):
                    dst = d0 if c < 8 else d1
                    dst[c % 8, pl.ds(j * LANES, LANES)] = vec

                vecs = [
                    plsc.load_gather(tab_v.at[c], [ivs[0]])
                    for c in range(HIDDEN_DIM)
                ]
                for j in range(8):
                    if j < 7:
                        nxt = []
                        for c in range(HIDDEN_DIM):
                            nxt.append(
                                plsc.load_gather(tab_v.at[c], [ivs[j + 1]])
                            )
                            store(j, c, vecs[c])
                        vecs = nxt
                    else:
                        for c in range(HIDDEN_DIM):
                            store(j, c, vecs[c])
                return carry

            lax.fori_loop(0, TPC, tile_blk, 0)

        def do_step(step, b):
            @pl.when(step >= NBUF)
            def _():
                wait_store(b)

            wait_idx(b)
            compute(b)
            start_store(step, b)

            @pl.when(step + NBUF < nk)
            def _():
                start_idx(step + NBUF, b)

        for b in range(NBUF):
            start_idx(b, b)

        def body(i, carry):
            do_step(2 * i, 0)
            do_step(2 * i + 1, 1)
            return carry

        lax.fori_loop(0, nk // 2, body, 0)

        @pl.when(nk % 2 == 1)
        def _():
            do_step(nk - 1, 0)

        for b in range(NBUF):
            wait_store(b)

    return gather_kernel


@functools.lru_cache(maxsize=1)
def _gather():
    return _make_gather()


def kernel(edge_attr, table):
    out_phys = _gather()(table.T, edge_attr)
    return out_phys.transpose(1, 3, 0, 2).reshape(N_EDGES, HIDDEN_DIM)
